# baseline (speedup 1.0000x reference)
import jax
import jax.numpy as jnp
from jax.experimental import pallas as pl
from jax.experimental.pallas import tpu as pltpu

_B, _T, _D = 16, 1024, 32
_K, _CD = 8192, 32
_M = 256
_NBLK = (_B * _T) // _M
_EPS = 1e-12


def _vq_step(z_ref, v_ref, g_ref, b_ref, fc_ref,
             zq_ref, idx_ref, loss_ref,
             cb_ref, c2_ref):
    i = pl.program_id(0)

    @pl.when(i == 0)
    def _init():
        v = v_ref[...]
        vn = jnp.sqrt(jnp.sum(v * v, axis=1, keepdims=True))
        w = g_ref[...].reshape(_D, 1) * v / vn
        cb = jnp.dot(fc_ref[...], w.T,
                     preferred_element_type=jnp.float32) + b_ref[...]
        cb_ref[...] = cb
        c2_ref[...] = jnp.sum(cb * cb, axis=1).reshape(1, _K)
        loss_ref[...] = jnp.zeros((1, 1), jnp.float32)

    cb = cb_ref[...]
    z = z_ref[...]
    z2 = jnp.sum(z * z, axis=1, keepdims=True)
    cross = jnp.dot(z, cb.T, preferred_element_type=jnp.float32)
    d2 = z2 - 2.0 * cross + c2_ref[...]

    dmin = jnp.min(d2, axis=1, keepdims=True)
    kiota = jax.lax.broadcasted_iota(jnp.int32, (_M, _K), 1)
    idx = jnp.min(jnp.where(d2 <= dmin, kiota, _K), axis=1)
    idx_ref[...] = idx.reshape(1, 1, _M)

    onehot = (kiota == idx.reshape(_M, 1)).astype(jnp.float32)
    zq = jnp.dot(onehot, cb, preferred_element_type=jnp.float32)

    diff = z - zq
    part = jnp.sum(diff * diff)
    loss_ref[...] += part.reshape(1, 1)

    norm_src = jnp.sqrt(jnp.sum(z * z, axis=1, keepdims=True))
    norm_tgt = jnp.sqrt(jnp.sum(zq * zq, axis=1, keepdims=True))
    u = z / jnp.maximum(norm_src, _EPS)
    q = zq / jnp.maximum(norm_tgt, _EPS)
    w_ = u + q
    wn = jnp.sqrt(jnp.sum(w_ * w_, axis=1, keepdims=True))
    w_ = w_ / jnp.maximum(wn, _EPS)
    rotated = (z
               - 2.0 * jnp.sum(z * w_, axis=1, keepdims=True) * w_
               + 2.0 * jnp.sum(z * u, axis=1, keepdims=True) * q)
    scale = norm_tgt / jnp.maximum(norm_src, _EPS)
    zq_ref[...] = rotated * scale


def kernel(z, v, g, b, frozen_codebook):
    zf = z.reshape(_B * _T, _D)
    g2 = g.reshape(1, _D)
    b2 = b.reshape(1, _D)

    grid_spec = pltpu.PrefetchScalarGridSpec(
        num_scalar_prefetch=0,
        grid=(_NBLK,),
        in_specs=[
            pl.BlockSpec((_M, _D), lambda i: (i, 0)),
            pl.BlockSpec((_D, _CD), lambda i: (0, 0)),
            pl.BlockSpec((1, _D), lambda i: (0, 0)),
            pl.BlockSpec((1, _D), lambda i: (0, 0)),
            pl.BlockSpec((_K, _CD), lambda i: (0, 0)),
        ],
        out_specs=[
            pl.BlockSpec((_M, _D), lambda i: (i, 0)),
            pl.BlockSpec((1, 1, _M), lambda i: (i, 0, 0)),
            pl.BlockSpec((1, 1), lambda i: (0, 0)),
        ],
        scratch_shapes=[
            pltpu.VMEM((_K, _D), jnp.float32),
            pltpu.VMEM((1, _K), jnp.float32),
        ],
    )

    zq, idx, loss = pl.pallas_call(
        _vq_step,
        grid_spec=grid_spec,
        out_shape=[
            jax.ShapeDtypeStruct((_B * _T, _D), jnp.float32),
            jax.ShapeDtypeStruct((_NBLK, 1, _M), jnp.int32),
            jax.ShapeDtypeStruct((1, 1), jnp.float32),
        ],
        compiler_params=pltpu.CompilerParams(
            dimension_semantics=("arbitrary",),
        ),
    )(zf, v, g2, b2, frozen_codebook)

    z_q = zq.reshape(_B, _T, _D)
    indices = idx.reshape(_B, _T)
    commit_loss = loss[0, 0] * (1.25 / (_B * _T * _D))
    return (z_q, indices, commit_loss)


# speedup vs baseline: 1.2377x; 1.2377x over previous
import jax
import jax.numpy as jnp
from jax.experimental import pallas as pl
from jax.experimental.pallas import tpu as pltpu

_B, _T, _D = 16, 1024, 32
_K, _CD = 8192, 32
_M = 256
_NBLK = (_B * _T) // _M
_EPS = 1e-12


def _vq_step(z_ref, v_ref, g_ref, b_ref, fc_ref,
             zq_ref, idx_ref, loss_ref,
             cb_ref, c2_ref):
    i = pl.program_id(0)

    @pl.when(i == 0)
    def _init():
        v = v_ref[...]
        vn = jnp.sqrt(jnp.sum(v * v, axis=1, keepdims=True))
        w = g_ref[...].reshape(_D, 1) * v / vn
        cb = jnp.dot(fc_ref[...], w.T,
                     preferred_element_type=jnp.float32) + b_ref[...]
        cb_ref[...] = cb
        c2 = jnp.sum(cb * cb, axis=1).reshape(1, _K)
        c2_ref[...] = jnp.broadcast_to(c2, (_M, _K))
        loss_ref[...] = jnp.zeros((1, 1), jnp.float32)

    cb = cb_ref[...]
    z = z_ref[...]
    z2 = jnp.sum(z * z, axis=1, keepdims=True)
    cross = jnp.dot(z, cb.T, preferred_element_type=jnp.float32)
    d2 = z2 - 2.0 * cross + c2_ref[...]

    idx = jnp.argmin(d2, axis=1).astype(jnp.int32)
    kiota = jax.lax.broadcasted_iota(jnp.int32, (_M, _K), 1)
    idx_ref[...] = idx.reshape(1, 1, _M)

    onehot = (kiota == idx.reshape(_M, 1)).astype(jnp.float32)
    zq = jnp.dot(onehot, cb, preferred_element_type=jnp.float32)

    diff = z - zq
    part = jnp.sum(diff * diff)
    loss_ref[...] += part.reshape(1, 1)

    norm_src = jnp.sqrt(jnp.sum(z * z, axis=1, keepdims=True))
    norm_tgt = jnp.sqrt(jnp.sum(zq * zq, axis=1, keepdims=True))
    u = z / jnp.maximum(norm_src, _EPS)
    q = zq / jnp.maximum(norm_tgt, _EPS)
    w_ = u + q
    wn = jnp.sqrt(jnp.sum(w_ * w_, axis=1, keepdims=True))
    w_ = w_ / jnp.maximum(wn, _EPS)
    rotated = (z
               - 2.0 * jnp.sum(z * w_, axis=1, keepdims=True) * w_
               + 2.0 * jnp.sum(z * u, axis=1, keepdims=True) * q)
    scale = norm_tgt / jnp.maximum(norm_src, _EPS)
    zq_ref[...] = rotated * scale


def kernel(z, v, g, b, frozen_codebook):
    zf = z.reshape(_B * _T, _D)
    g2 = g.reshape(1, _D)
    b2 = b.reshape(1, _D)

    grid_spec = pltpu.PrefetchScalarGridSpec(
        num_scalar_prefetch=0,
        grid=(_NBLK,),
        in_specs=[
            pl.BlockSpec((_M, _D), lambda i: (i, 0)),
            pl.BlockSpec((_D, _CD), lambda i: (0, 0)),
            pl.BlockSpec((1, _D), lambda i: (0, 0)),
            pl.BlockSpec((1, _D), lambda i: (0, 0)),
            pl.BlockSpec((_K, _CD), lambda i: (0, 0)),
        ],
        out_specs=[
            pl.BlockSpec((_M, _D), lambda i: (i, 0)),
            pl.BlockSpec((1, 1, _M), lambda i: (i, 0, 0)),
            pl.BlockSpec((1, 1), lambda i: (0, 0)),
        ],
        scratch_shapes=[
            pltpu.VMEM((_K, _D), jnp.float32),
            pltpu.VMEM((_M, _K), jnp.float32),
        ],
    )

    zq, idx, loss = pl.pallas_call(
        _vq_step,
        grid_spec=grid_spec,
        out_shape=[
            jax.ShapeDtypeStruct((_B * _T, _D), jnp.float32),
            jax.ShapeDtypeStruct((_NBLK, 1, _M), jnp.int32),
            jax.ShapeDtypeStruct((1, 1), jnp.float32),
        ],
        compiler_params=pltpu.CompilerParams(
            dimension_semantics=("arbitrary",),
        ),
    )(zf, v, g2, b2, frozen_codebook)

    z_q = zq.reshape(_B, _T, _D)
    indices = idx.reshape(_B, _T)
    commit_loss = loss[0, 0] * (1.25 / (_B * _T * _D))
    return (z_q, indices, commit_loss)
